# baseline (speedup 1.0000x reference)
import functools

import jax
import jax.numpy as jnp
from jax.experimental import pallas as pl
from jax.experimental.pallas import tpu as pltpu

_N = 17
_D = 64
_EPS = 1e-3


def _fused_block(x_ref, at_ref, w0_ref, t0_ref, w1_ref, t1_ref, o_ref):
    bb = x_ref.shape[0]
    x = x_ref[...]
    at = at_ref[...]
    h = x
    for w_ref, t_ref in ((w0_ref, t0_ref), (w1_ref, t1_ref)):
        hd = jnp.swapaxes(h, 1, 2).reshape(bb * _D, _N)
        y = jnp.dot(hd, at, preferred_element_type=jnp.float32)
        y = jnp.swapaxes(y.reshape(bb, _D, _N), 1, 2)
        y = y.reshape(bb * _N, _D)
        z = jnp.dot(y, w_ref[...], preferred_element_type=jnp.float32)
        z = jnp.maximum(z + t_ref[...], 0.0)
        h = z.reshape(bb, _N, _D)
    o_ref[...] = h + x


@functools.partial(jax.jit, static_argnames=("block_b",))
def _run(inputs, affinity, w0p, t0, w1p, t1, block_b):
    b = inputs.shape[0]
    grid = (b // block_b,)
    return pl.pallas_call(
        _fused_block,
        grid=grid,
        in_specs=[
            pl.BlockSpec((block_b, _N, _D), lambda i: (i, 0, 0)),
            pl.BlockSpec((_N, _N), lambda i: (0, 0)),
            pl.BlockSpec((_D, _D), lambda i: (0, 0)),
            pl.BlockSpec((1, _D), lambda i: (0, 0)),
            pl.BlockSpec((_D, _D), lambda i: (0, 0)),
            pl.BlockSpec((1, _D), lambda i: (0, 0)),
        ],
        out_specs=pl.BlockSpec((block_b, _N, _D), lambda i: (i, 0, 0)),
        out_shape=jax.ShapeDtypeStruct((b, _N, _D), jnp.float32),
    )(inputs, affinity, w0p, t0, w1p, t1)


def kernel(inputs, affinity, W0, b0, gamma0, beta0, mean0, var0,
           W1, b1, gamma1, beta1, mean1, var1):
    s0 = gamma0 * jax.lax.rsqrt(var0 + _EPS)
    s1 = gamma1 * jax.lax.rsqrt(var1 + _EPS)
    w0p = W0 * s0[None, :]
    w1p = W1 * s1[None, :]
    t0 = (b0 * s0 + beta0 - mean0 * s0).reshape(1, _D)
    t1 = (b1 * s1 + beta1 - mean1 * s1).reshape(1, _D)
    return _run(inputs, affinity.T, w0p, t0, w1p, t1, block_b=512)


# speedup vs baseline: 1.9735x; 1.9735x over previous
import functools

import jax
import jax.numpy as jnp
from jax.experimental import pallas as pl
from jax.experimental.pallas import tpu as pltpu

_N = 17
_D = 64
_EPS = 1e-3


_NP = 24


def _fused_block(x_ref, a_ref, w0_ref, t0_ref, w1_ref, t1_ref, o_ref):
    bb = x_ref.shape[0]
    x = x_ref[...]
    a = a_ref[...]
    y0 = jax.lax.dot_general(a, x, (((1,), (1,)), ((), ())),
                             preferred_element_type=jnp.float32)
    z0 = jnp.dot(y0.reshape(_N * bb, _D), w0_ref[...],
                 preferred_element_type=jnp.float32).reshape(_N, bb, _D)
    h1 = jnp.maximum(z0 + t0_ref[...], 0.0)
    y1 = jax.lax.dot_general(a, h1, (((1,), (0,)), ((), ())),
                             preferred_element_type=jnp.float32)
    z1 = jnp.dot(y1.reshape(_N * bb, _D), w1_ref[...],
                 preferred_element_type=jnp.float32).reshape(_N, bb, _D)
    h2 = jnp.maximum(z1 + t1_ref[...], 0.0)
    o_ref[...] = jnp.swapaxes(h2, 0, 1) + x


@functools.partial(jax.jit, static_argnames=("block_b",))
def _run(inputs, affinity, w0p, t0, w1p, t1, block_b):
    b = inputs.shape[0]
    grid = (b // block_b,)
    return pl.pallas_call(
        _fused_block,
        grid=grid,
        in_specs=[
            pl.BlockSpec((block_b, _N, _D), lambda i: (i, 0, 0)),
            pl.BlockSpec((_N, _N), lambda i: (0, 0)),
            pl.BlockSpec((_D, _D), lambda i: (0, 0)),
            pl.BlockSpec((1, 1, _D), lambda i: (0, 0, 0)),
            pl.BlockSpec((_D, _D), lambda i: (0, 0)),
            pl.BlockSpec((1, 1, _D), lambda i: (0, 0, 0)),
        ],
        out_specs=pl.BlockSpec((block_b, _N, _D), lambda i: (i, 0, 0)),
        out_shape=jax.ShapeDtypeStruct((b, _N, _D), jnp.float32),
    )(inputs, affinity, w0p, t0, w1p, t1)


def kernel(inputs, affinity, W0, b0, gamma0, beta0, mean0, var0,
           W1, b1, gamma1, beta1, mean1, var1):
    s0 = gamma0 * jax.lax.rsqrt(var0 + _EPS)
    s1 = gamma1 * jax.lax.rsqrt(var1 + _EPS)
    w0p = W0 * s0[None, :]
    w1p = W1 * s1[None, :]
    t0 = (b0 * s0 + beta0 - mean0 * s0).reshape(1, 1, _D)
    t1 = (b1 * s1 + beta1 - mean1 * s1).reshape(1, 1, _D)
    return _run(inputs, affinity, w0p, t0, w1p, t1, block_b=512)


# speedup vs baseline: 9.9718x; 5.0528x over previous
import functools

import jax
import jax.numpy as jnp
from jax.experimental import pallas as pl

_N = 17
_D = 64
_EPS = 1e-3


def _fused_block(x_ref, a_ref, w0t_ref, t0_ref, w1t_ref, t1_ref, o_ref):
    bb = x_ref.shape[-1]
    x = x_ref[...]
    a = a_ref[...]
    h = x
    for wt_ref, t_ref in ((w0t_ref, t0_ref), (w1t_ref, t1_ref)):
        y = jnp.concatenate(
            [jax.lax.dot_general(a, h[:, :, c * 128:(c + 1) * 128],
                                 (((1,), (0,)), ((), ())),
                                 preferred_element_type=jnp.float32)
             for c in range(bb // 128)], axis=-1)
        wt = wt_ref[...]
        z = jnp.stack(
            [jnp.dot(wt, y[j], preferred_element_type=jnp.float32)
             for j in range(_N)], axis=0)
        h = jnp.maximum(z + t_ref[...], 0.0)
    o_ref[...] = h + x


@functools.partial(jax.jit, static_argnames=("block_b",))
def _run(xt, affinity, w0t, t0, w1t, t1, block_b):
    b = xt.shape[-1]
    grid = (b // block_b,)
    return pl.pallas_call(
        _fused_block,
        grid=grid,
        in_specs=[
            pl.BlockSpec((_N, _D, block_b), lambda i: (0, 0, i)),
            pl.BlockSpec((_N, _N), lambda i: (0, 0)),
            pl.BlockSpec((_D, _D), lambda i: (0, 0)),
            pl.BlockSpec((1, _D, 1), lambda i: (0, 0, 0)),
            pl.BlockSpec((_D, _D), lambda i: (0, 0)),
            pl.BlockSpec((1, _D, 1), lambda i: (0, 0, 0)),
        ],
        out_specs=pl.BlockSpec((_N, _D, block_b), lambda i: (0, 0, i)),
        out_shape=jax.ShapeDtypeStruct((_N, _D, b), jnp.float32),
    )(xt, affinity, w0t, t0, w1t, t1)


def kernel(inputs, affinity, W0, b0, gamma0, beta0, mean0, var0,
           W1, b1, gamma1, beta1, mean1, var1):
    s0 = gamma0 * jax.lax.rsqrt(var0 + _EPS)
    s1 = gamma1 * jax.lax.rsqrt(var1 + _EPS)
    w0t = (W0 * s0[None, :]).T
    w1t = (W1 * s1[None, :]).T
    t0 = (b0 * s0 + beta0 - mean0 * s0).reshape(1, _D, 1)
    t1 = (b1 * s1 + beta1 - mean1 * s1).reshape(1, _D, 1)
    xt = jnp.transpose(inputs, (1, 2, 0))
    out = _run(xt, affinity, w0t, t0, w1t, t1, block_b=512)
    return jnp.transpose(out, (2, 0, 1))


# speedup vs baseline: 10.3244x; 1.0354x over previous
import functools

import jax
import jax.numpy as jnp
from jax.experimental import pallas as pl

_N = 17
_D = 64
_EPS = 1e-3


def _fused_block(x_ref, a_ref, w0t_ref, t0_ref, w1t_ref, t1_ref, o_ref):
    bb = x_ref.shape[-1]
    x = x_ref[...]
    a = a_ref[...].astype(jnp.bfloat16)
    h = x
    for wt_ref, t_ref in ((w0t_ref, t0_ref), (w1t_ref, t1_ref)):
        hb = h.astype(jnp.bfloat16)
        y = jnp.concatenate(
            [jax.lax.dot_general(a, hb[:, :, c * 128:(c + 1) * 128],
                                 (((1,), (0,)), ((), ())),
                                 preferred_element_type=jnp.float32)
             for c in range(bb // 128)], axis=-1)
        wt = wt_ref[...].astype(jnp.bfloat16)
        z = jnp.stack(
            [jnp.dot(wt, y[j].astype(jnp.bfloat16),
                     preferred_element_type=jnp.float32)
             for j in range(_N)], axis=0)
        h = jnp.maximum(z + t_ref[...], 0.0)
    o_ref[...] = h + x


@functools.partial(jax.jit, static_argnames=("block_b",))
def _run(xt, affinity, w0t, t0, w1t, t1, block_b):
    b = xt.shape[-1]
    grid = (b // block_b,)
    return pl.pallas_call(
        _fused_block,
        grid=grid,
        in_specs=[
            pl.BlockSpec((_N, _D, block_b), lambda i: (0, 0, i)),
            pl.BlockSpec((_N, _N), lambda i: (0, 0)),
            pl.BlockSpec((_D, _D), lambda i: (0, 0)),
            pl.BlockSpec((1, _D, 1), lambda i: (0, 0, 0)),
            pl.BlockSpec((_D, _D), lambda i: (0, 0)),
            pl.BlockSpec((1, _D, 1), lambda i: (0, 0, 0)),
        ],
        out_specs=pl.BlockSpec((_N, _D, block_b), lambda i: (0, 0, i)),
        out_shape=jax.ShapeDtypeStruct((_N, _D, b), jnp.float32),
    )(xt, affinity, w0t, t0, w1t, t1)


def kernel(inputs, affinity, W0, b0, gamma0, beta0, mean0, var0,
           W1, b1, gamma1, beta1, mean1, var1):
    s0 = gamma0 * jax.lax.rsqrt(var0 + _EPS)
    s1 = gamma1 * jax.lax.rsqrt(var1 + _EPS)
    w0t = (W0 * s0[None, :]).T
    w1t = (W1 * s1[None, :]).T
    t0 = (b0 * s0 + beta0 - mean0 * s0).reshape(1, _D, 1)
    t1 = (b1 * s1 + beta1 - mean1 * s1).reshape(1, _D, 1)
    xt = jnp.transpose(inputs, (1, 2, 0))
    out = _run(xt, affinity, w0t, t0, w1t, t1, block_b=512)
    return jnp.transpose(out, (2, 0, 1))
